# baseline (device time: 16098 ns/iter reference)
import jax
import jax.numpy as jnp
from jax import lax
from jax.experimental import pallas as pl
from jax.experimental.pallas import tpu as pltpu

N_CHUNKS = 8


def kernel(A, B):
    m, k = A.shape
    _, n = B.shape
    half = m // 2
    rc = half // N_CHUNKS

    def body(a_ref, b_ref, out_ref,
             a_vmem, b_vmem, acc_ref, xrecv_ref, red_ref,
             in_sems, out_sems,
             x_send_sems, x_recv_sems, y_send_sems, y_recv_sems):
        my_x = lax.axis_index("x")
        my_y = lax.axis_index("y")
        x_nbr = (1 - my_x, my_y)
        y_nbr = (my_x, 1 - my_y)

        my_base = my_y * half

        a_in = pltpu.make_async_copy(
            a_ref.at[pl.ds(my_base, half), :], a_vmem, in_sems.at[0]
        )
        b_in = pltpu.make_async_copy(b_ref, b_vmem, in_sems.at[1])
        a_in.start()
        b_in.start()

        barrier_sem = pltpu.get_barrier_semaphore()
        for nbr in (x_nbr, y_nbr):
            pl.semaphore_signal(
                barrier_sem, inc=1, device_id=nbr,
                device_id_type=pl.DeviceIdType.MESH,
            )
        pl.semaphore_wait(barrier_sem, 2)

        a_in.wait()
        b_in.wait()

        x_rdmas = []
        for c in range(N_CHUNKS):
            rows = pl.ds(c * rc, rc)
            acc_ref[rows, :] = jnp.dot(
                a_vmem[rows, :], b_vmem[:, :],
                preferred_element_type=jnp.float32,
            )
            rdma = pltpu.make_async_remote_copy(
                src_ref=acc_ref.at[rows, :],
                dst_ref=xrecv_ref.at[rows, :],
                send_sem=x_send_sems.at[c],
                recv_sem=x_recv_sems.at[c],
                device_id=x_nbr,
                device_id_type=pl.DeviceIdType.MESH,
            )
            rdma.start()
            x_rdmas.append(rdma)

        y_rdmas = []
        out_copies = []
        for c in range(N_CHUNKS):
            rows = pl.ds(c * rc, rc)
            out_rows = pl.ds(my_base + c * rc, rc)
            x_rdmas[c].wait_recv()
            red_ref[rows, :] = acc_ref[rows, :] + xrecv_ref[rows, :]
            rdma = pltpu.make_async_remote_copy(
                src_ref=red_ref.at[rows, :],
                dst_ref=out_ref.at[out_rows, :],
                send_sem=y_send_sems.at[c],
                recv_sem=y_recv_sems.at[c],
                device_id=y_nbr,
                device_id_type=pl.DeviceIdType.MESH,
            )
            rdma.start()
            y_rdmas.append(rdma)
            cp = pltpu.make_async_copy(
                red_ref.at[rows, :], out_ref.at[out_rows, :], out_sems.at[c]
            )
            cp.start()
            out_copies.append(cp)

        for c in range(N_CHUNKS):
            y_rdmas[c].wait_recv()
            out_copies[c].wait()
        for c in range(N_CHUNKS):
            x_rdmas[c].wait_send()
            y_rdmas[c].wait_send()

    return pl.pallas_call(
        body,
        out_shape=jax.ShapeDtypeStruct((m, n), jnp.float32),
        in_specs=[
            pl.BlockSpec(memory_space=pltpu.MemorySpace.HBM),
            pl.BlockSpec(memory_space=pltpu.MemorySpace.HBM),
        ],
        out_specs=pl.BlockSpec(memory_space=pltpu.MemorySpace.HBM),
        scratch_shapes=[
            pltpu.VMEM((half, k), jnp.float32),
            pltpu.VMEM((k, n), jnp.float32),
            pltpu.VMEM((half, n), jnp.float32),
            pltpu.VMEM((half, n), jnp.float32),
            pltpu.VMEM((half, n), jnp.float32),
            pltpu.SemaphoreType.DMA((2,)),
            pltpu.SemaphoreType.DMA((N_CHUNKS,)),
            pltpu.SemaphoreType.DMA((N_CHUNKS,)),
            pltpu.SemaphoreType.DMA((N_CHUNKS,)),
            pltpu.SemaphoreType.DMA((N_CHUNKS,)),
            pltpu.SemaphoreType.DMA((N_CHUNKS,)),
        ],
        compiler_params=pltpu.CompilerParams(collective_id=0),
    )(A, B)
